# Initial kernel scaffold; baseline (speedup 1.0000x reference)
import jax
import jax.numpy as jnp
from jax.experimental import pallas as pl


def kernel(x, frame_indices):
    raise NotImplementedError("write your pallas kernel here")


# speedup vs baseline: 30.8476x; 30.8476x over previous
import functools

import jax
import jax.numpy as jnp
from jax import lax
from jax.experimental import pallas as pl
from jax.experimental.pallas import tpu as pltpu
from jax.experimental.pallas import tpu_sc as plsc

_LANES = 16
_HAND_LO = 33 * 3
_HAND_HI = 54 * 3


def _mask_body(n_pad, frames_per_w, idx_hbm, mask_hbm, idx_v, mask_v):
    num_cores = 2
    wid = lax.axis_index("s") * num_cores + lax.axis_index("c")
    lo = wid * frames_per_w

    pltpu.sync_copy(idx_hbm, idx_v)

    zeros = jnp.zeros((_LANES,), jnp.int32)

    def zero_body(i, carry):
        mask_v[pl.ds(i * _LANES, _LANES)] = zeros
        return carry

    lax.fori_loop(0, frames_per_w // _LANES, zero_body, 0)

    ones = jnp.ones((_LANES,), jnp.int32)

    def scan_body(i, carry):
        v = idx_v[pl.ds(i * _LANES, _LANES)]
        rel = v - lo
        inb = (rel >= 0) & (rel < frames_per_w)
        relc = jnp.clip(rel, 0, frames_per_w - 1)
        plsc.store_scatter(mask_v, [relc], ones, mask=inb)
        return carry

    lax.fori_loop(0, n_pad // _LANES, scan_body, 0)

    pltpu.sync_copy(mask_v, mask_hbm.at[pl.ds(lo, frames_per_w)])


def _build_mask(idx_padded, t):
    n_pad = idx_padded.shape[0]
    num_workers = 32
    frames_per_w = t // num_workers
    mesh = plsc.VectorSubcoreMesh(core_axis_name="c", subcore_axis_name="s")
    return pl.kernel(
        functools.partial(_mask_body, n_pad, frames_per_w),
        out_type=jax.ShapeDtypeStruct((t,), jnp.int32),
        mesh=mesh,
        scratch_types=[
            pltpu.VMEM((n_pad,), jnp.int32),
            pltpu.VMEM((frames_per_w,), jnp.int32),
        ],
        compiler_params=pltpu.CompilerParams(needs_layout_passes=False),
    )(idx_padded)


def _apply_body(mask_ref, x_ref, o_ref):
    rows, cols = x_ref.shape
    m = mask_ref[0, 0, :]
    col = lax.broadcasted_iota(jnp.int32, (rows, cols), 1)
    hand = (col >= _HAND_LO) & (col < _HAND_HI)
    sel = hand & (m[:, None] != 0)
    o_ref[...] = jnp.where(sel, jnp.float32(jnp.nan), x_ref[...])


def kernel(x, frame_indices):
    t, num_landmarks, coords = x.shape
    row = num_landmarks * coords
    n = frame_indices.shape[0]

    n_pad = ((n + _LANES - 1) // _LANES) * _LANES
    if n_pad != n:
        idx_padded = jnp.concatenate(
            [frame_indices, jnp.broadcast_to(frame_indices[:1], (n_pad - n,))]
        )
    else:
        idx_padded = frame_indices

    mask = _build_mask(idx_padded, t)

    block_rows = 2048
    grid = t // block_rows
    x2 = x.reshape(t, row)
    out = pl.pallas_call(
        _apply_body,
        grid=(grid,),
        in_specs=[
            pl.BlockSpec((1, 1, block_rows), lambda i: (i, 0, 0)),
            pl.BlockSpec((block_rows, row), lambda i: (i, 0)),
        ],
        out_specs=pl.BlockSpec((block_rows, row), lambda i: (i, 0)),
        out_shape=jax.ShapeDtypeStruct((t, row), jnp.float32),
    )(mask.reshape(grid, 1, block_rows), x2)
    return out.reshape(t, num_landmarks, coords)


# speedup vs baseline: 290.4059x; 9.4142x over previous
import functools

import jax
import jax.numpy as jnp
from jax import lax
from jax.experimental import pallas as pl
from jax.experimental.pallas import tpu as pltpu
from jax.experimental.pallas import tpu_sc as plsc

_LANES = 16
_HAND_LO = 33 * 3
_HAND_HI = 54 * 3


def _mask_body(n_pad, frames_per_w, idx_hbm, mask_hbm, idx_v, mask_v):
    num_cores = 2
    wid = lax.axis_index("s") * num_cores + lax.axis_index("c")
    lo = wid * frames_per_w

    pltpu.sync_copy(idx_hbm, idx_v)

    zeros = jnp.zeros((_LANES,), jnp.int32)

    def zero_body(i, carry):
        mask_v[pl.ds(i * _LANES, _LANES)] = zeros
        return carry

    lax.fori_loop(0, frames_per_w // _LANES, zero_body, 0)

    ones = jnp.ones((_LANES,), jnp.int32)

    def scan_body(i, carry):
        v = idx_v[pl.ds(i * _LANES, _LANES)]
        rel = v - lo
        inb = (rel >= 0) & (rel < frames_per_w)
        relc = jnp.clip(rel, 0, frames_per_w - 1)
        plsc.store_scatter(mask_v, [relc], ones, mask=inb)
        return carry

    lax.fori_loop(0, n_pad // _LANES, scan_body, 0)

    pltpu.sync_copy(mask_v, mask_hbm.at[pl.ds(lo, frames_per_w)])


def _build_mask(idx_padded, t):
    n_pad = idx_padded.shape[0]
    num_workers = 32
    frames_per_w = t // num_workers
    mesh = plsc.VectorSubcoreMesh(core_axis_name="c", subcore_axis_name="s")
    return pl.kernel(
        functools.partial(_mask_body, n_pad, frames_per_w),
        out_type=jax.ShapeDtypeStruct((t,), jnp.int32),
        mesh=mesh,
        scratch_types=[
            pltpu.VMEM((n_pad,), jnp.int32),
            pltpu.VMEM((frames_per_w,), jnp.int32),
        ],
        compiler_params=pltpu.CompilerParams(needs_layout_passes=False),
    )(idx_padded)


def _apply_body(mask_ref, x_ref, o_ref):
    shape = x_ref.shape
    m = mask_ref[0, 0, :]
    lmk = lax.broadcasted_iota(jnp.int32, shape, 1)
    hand = (lmk >= 33) & (lmk < 54)
    sel = hand & (m != 0)[None, None, :]
    o_ref[...] = jnp.where(sel, jnp.float32(jnp.nan), x_ref[...])


def kernel(x, frame_indices):
    t, num_landmarks, coords = x.shape
    row = num_landmarks * coords
    n = frame_indices.shape[0]

    n_pad = ((n + _LANES - 1) // _LANES) * _LANES
    if n_pad != n:
        idx_padded = jnp.concatenate(
            [frame_indices, jnp.broadcast_to(frame_indices[:1], (n_pad - n,))]
        )
    else:
        idx_padded = frame_indices

    mask = _build_mask(idx_padded, t)

    blk = 2048
    grid = t // blk
    xt = jnp.transpose(x, (2, 1, 0))
    out = pl.pallas_call(
        _apply_body,
        grid=(grid,),
        in_specs=[
            pl.BlockSpec((1, 1, blk), lambda i: (i, 0, 0)),
            pl.BlockSpec((coords, num_landmarks, blk), lambda i: (0, 0, i)),
        ],
        out_specs=pl.BlockSpec((coords, num_landmarks, blk), lambda i: (0, 0, i)),
        out_shape=jax.ShapeDtypeStruct((coords, num_landmarks, t), jnp.float32),
    )(mask.reshape(grid, 1, blk), xt)
    return jnp.transpose(out, (2, 1, 0))


# speedup vs baseline: 312.0518x; 1.0745x over previous
import functools

import jax
import jax.numpy as jnp
from jax import lax
from jax.experimental import pallas as pl
from jax.experimental.pallas import tpu as pltpu
from jax.experimental.pallas import tpu_sc as plsc

_LANES = 16
_HAND_LO = 33 * 3
_HAND_HI = 54 * 3


def _mask_body(n_pad, frames_per_w, idx_hbm, mask_hbm, idx_v, mask_v):
    num_cores = 2
    wid = lax.axis_index("s") * num_cores + lax.axis_index("c")
    lo = wid * frames_per_w

    pltpu.sync_copy(idx_hbm, idx_v)

    zeros = jnp.zeros((_LANES,), jnp.int32)

    def zero_body(i, carry):
        mask_v[pl.ds(i * _LANES, _LANES)] = zeros
        return carry

    lax.fori_loop(0, frames_per_w // _LANES, zero_body, 0)

    ones = jnp.ones((_LANES,), jnp.int32)

    def scan_body(i, carry):
        v = idx_v[pl.ds(i * _LANES, _LANES)]
        rel = v - lo
        inb = (rel >= 0) & (rel < frames_per_w)
        relc = jnp.clip(rel, 0, frames_per_w - 1)
        plsc.store_scatter(mask_v, [relc], ones, mask=inb)
        return carry

    lax.fori_loop(0, n_pad // _LANES, scan_body, 0)

    pltpu.sync_copy(mask_v, mask_hbm.at[pl.ds(lo, frames_per_w)])


def _build_mask(idx_padded, t):
    n_pad = idx_padded.shape[0]
    num_workers = 32
    frames_per_w = t // num_workers
    mesh = plsc.VectorSubcoreMesh(core_axis_name="c", subcore_axis_name="s")
    return pl.kernel(
        functools.partial(_mask_body, n_pad, frames_per_w),
        out_type=jax.ShapeDtypeStruct((t,), jnp.int32),
        mesh=mesh,
        scratch_types=[
            pltpu.VMEM((n_pad,), jnp.int32),
            pltpu.VMEM((frames_per_w,), jnp.int32),
        ],
        compiler_params=pltpu.CompilerParams(needs_layout_passes=False),
    )(idx_padded)


def _apply_body(mask_ref, x_ref, o_ref):
    shape = x_ref.shape
    m = mask_ref[0, 0, :]
    lmk = lax.broadcasted_iota(jnp.int32, shape, 1)
    hand = (lmk >= 33) & (lmk < 54)
    sel = hand & (m != 0)[None, None, :]
    o_ref[...] = jnp.where(sel, jnp.float32(jnp.nan), x_ref[...])


def kernel(x, frame_indices):
    t, num_landmarks, coords = x.shape
    row = num_landmarks * coords
    n = frame_indices.shape[0]

    n_pad = ((n + _LANES - 1) // _LANES) * _LANES
    if n_pad != n:
        idx_padded = jnp.concatenate(
            [frame_indices, jnp.broadcast_to(frame_indices[:1], (n_pad - n,))]
        )
    else:
        idx_padded = frame_indices

    mask = _build_mask(idx_padded, t)

    blk = 4096
    grid = t // blk
    xt = jnp.transpose(x, (2, 1, 0))
    out = pl.pallas_call(
        _apply_body,
        grid=(grid,),
        in_specs=[
            pl.BlockSpec((1, 1, blk), lambda i: (i, 0, 0)),
            pl.BlockSpec((coords, num_landmarks, blk), lambda i: (0, 0, i)),
        ],
        out_specs=pl.BlockSpec((coords, num_landmarks, blk), lambda i: (0, 0, i)),
        out_shape=jax.ShapeDtypeStruct((coords, num_landmarks, t), jnp.float32),
    )(mask.reshape(grid, 1, blk), xt)
    return jnp.transpose(out, (2, 1, 0))


# speedup vs baseline: 321.4161x; 1.0300x over previous
import functools

import jax
import jax.numpy as jnp
from jax import lax
from jax.experimental import pallas as pl
from jax.experimental.pallas import tpu as pltpu
from jax.experimental.pallas import tpu_sc as plsc

_LANES = 16
_HAND_LO = 33 * 3
_HAND_HI = 54 * 3


def _mask_body(n_pad, frames_per_w, idx_hbm, mask_hbm, idx_v, mask_v):
    num_cores = 2
    wid = lax.axis_index("s") * num_cores + lax.axis_index("c")
    lo = wid * frames_per_w

    pltpu.sync_copy(idx_hbm, idx_v)

    zeros = jnp.zeros((_LANES,), jnp.int32)

    def zero_body(i, carry):
        mask_v[pl.ds(i * _LANES, _LANES)] = zeros
        return carry

    lax.fori_loop(0, frames_per_w // _LANES, zero_body, 0)

    ones = jnp.ones((_LANES,), jnp.int32)

    def scan_body(i, carry):
        v = idx_v[pl.ds(i * _LANES, _LANES)]
        rel = v - lo
        inb = (rel >= 0) & (rel < frames_per_w)
        relc = jnp.clip(rel, 0, frames_per_w - 1)
        plsc.store_scatter(mask_v, [relc], ones, mask=inb)
        return carry

    lax.fori_loop(0, n_pad // _LANES, scan_body, 0)

    pltpu.sync_copy(mask_v, mask_hbm.at[pl.ds(lo, frames_per_w)])


def _build_mask(idx_padded, t):
    n_pad = idx_padded.shape[0]
    num_workers = 32
    frames_per_w = t // num_workers
    mesh = plsc.VectorSubcoreMesh(core_axis_name="c", subcore_axis_name="s")
    return pl.kernel(
        functools.partial(_mask_body, n_pad, frames_per_w),
        out_type=jax.ShapeDtypeStruct((t,), jnp.int32),
        mesh=mesh,
        scratch_types=[
            pltpu.VMEM((n_pad,), jnp.int32),
            pltpu.VMEM((frames_per_w,), jnp.int32),
        ],
        compiler_params=pltpu.CompilerParams(needs_layout_passes=False),
    )(idx_padded)


def _apply_body(mask_ref, x_ref, o_ref):
    shape = x_ref.shape
    m = mask_ref[0, 0, :]
    lmk = lax.broadcasted_iota(jnp.int32, shape, 1)
    hand = (lmk >= 33) & (lmk < 54)
    sel = hand & (m != 0)[None, None, :]
    o_ref[...] = jnp.where(sel, jnp.float32(jnp.nan), x_ref[...])


def kernel(x, frame_indices):
    t, num_landmarks, coords = x.shape
    row = num_landmarks * coords
    n = frame_indices.shape[0]

    n_pad = ((n + _LANES - 1) // _LANES) * _LANES
    if n_pad != n:
        idx_padded = jnp.concatenate(
            [frame_indices, jnp.broadcast_to(frame_indices[:1], (n_pad - n,))]
        )
    else:
        idx_padded = frame_indices

    mask = _build_mask(idx_padded, t)

    blk = 8192
    grid = t // blk
    xt = jnp.transpose(x, (2, 1, 0))
    out = pl.pallas_call(
        _apply_body,
        grid=(grid,),
        in_specs=[
            pl.BlockSpec((1, 1, blk), lambda i: (i, 0, 0)),
            pl.BlockSpec((coords, num_landmarks, blk), lambda i: (0, 0, i)),
        ],
        out_specs=pl.BlockSpec((coords, num_landmarks, blk), lambda i: (0, 0, i)),
        out_shape=jax.ShapeDtypeStruct((coords, num_landmarks, t), jnp.float32),
    )(mask.reshape(grid, 1, blk), xt)
    return jnp.transpose(out, (2, 1, 0))


# speedup vs baseline: 344.5923x; 1.0721x over previous
import functools

import jax
import jax.numpy as jnp
from jax import lax
from jax.experimental import pallas as pl
from jax.experimental.pallas import tpu as pltpu
from jax.experimental.pallas import tpu_sc as plsc

_LANES = 16
_HAND_LO = 33 * 3
_HAND_HI = 54 * 3


def _mask_body(n_pad, frames_per_w, idx_hbm, mask_hbm, idx_v, mask_v):
    num_cores = 2
    wid = lax.axis_index("s") * num_cores + lax.axis_index("c")
    lo = wid * frames_per_w

    pltpu.sync_copy(idx_hbm, idx_v)

    zeros = jnp.zeros((_LANES,), jnp.int32)

    @plsc.parallel_loop(0, frames_per_w, step=_LANES, unroll=8)
    def _zero(i):
        mask_v[pl.ds(i, _LANES)] = zeros

    ones = jnp.ones((_LANES,), jnp.int32)

    @plsc.parallel_loop(0, n_pad, step=_LANES, unroll=8)
    def _scan(i):
        v = idx_v[pl.ds(i, _LANES)]
        rel = v - lo
        inb = (rel >= 0) & (rel < frames_per_w)
        relc = jnp.clip(rel, 0, frames_per_w - 1)
        plsc.store_scatter(mask_v, [relc], ones, mask=inb)

    pltpu.sync_copy(mask_v, mask_hbm.at[pl.ds(lo, frames_per_w)])


def _build_mask(idx_padded, t):
    n_pad = idx_padded.shape[0]
    num_workers = 32
    frames_per_w = t // num_workers
    mesh = plsc.VectorSubcoreMesh(core_axis_name="c", subcore_axis_name="s")
    return pl.kernel(
        functools.partial(_mask_body, n_pad, frames_per_w),
        out_type=jax.ShapeDtypeStruct((t,), jnp.int32),
        mesh=mesh,
        scratch_types=[
            pltpu.VMEM((n_pad,), jnp.int32),
            pltpu.VMEM((frames_per_w,), jnp.int32),
        ],
        compiler_params=pltpu.CompilerParams(needs_layout_passes=False),
    )(idx_padded)


def _apply_body(mask_ref, x_ref, o_ref):
    shape = x_ref.shape
    m = mask_ref[0, 0, :]
    lmk = lax.broadcasted_iota(jnp.int32, shape, 1)
    hand = (lmk >= 33) & (lmk < 54)
    sel = hand & (m != 0)[None, None, :]
    o_ref[...] = jnp.where(sel, jnp.float32(jnp.nan), x_ref[...])


def kernel(x, frame_indices):
    t, num_landmarks, coords = x.shape
    row = num_landmarks * coords
    n = frame_indices.shape[0]

    chunk = _LANES * 8
    n_pad = ((n + chunk - 1) // chunk) * chunk
    if n_pad != n:
        idx_padded = jnp.concatenate(
            [frame_indices, jnp.broadcast_to(frame_indices[:1], (n_pad - n,))]
        )
    else:
        idx_padded = frame_indices

    mask = _build_mask(idx_padded, t)

    blk = 8192
    grid = t // blk
    xt = jnp.transpose(x, (2, 1, 0))
    out = pl.pallas_call(
        _apply_body,
        grid=(grid,),
        in_specs=[
            pl.BlockSpec((1, 1, blk), lambda i: (i, 0, 0)),
            pl.BlockSpec((coords, num_landmarks, blk), lambda i: (0, 0, i)),
        ],
        out_specs=pl.BlockSpec((coords, num_landmarks, blk), lambda i: (0, 0, i)),
        out_shape=jax.ShapeDtypeStruct((coords, num_landmarks, t), jnp.float32),
    )(mask.reshape(grid, 1, blk), xt)
    return jnp.transpose(out, (2, 1, 0))
